# baseline (speedup 1.0000x reference)
import functools

import jax
import jax.numpy as jnp
from jax import lax
from jax.experimental import pallas as pl
from jax.experimental.pallas import tpu as pltpu
from jax.experimental.pallas import tpu_sc as plsc

NN = 10000
DIN = 128
DH = 256
DOUT = 64
EDGES = 320000
NC, NS = 2, 16
CHUNK = 112
ACCR = 10112
TROWS = ACCR // NS
EPAD = 322560
NROWCH = EPAD // CHUNK
CH_AGG = NROWCH // NS
CH_DEG = NROWCH // (NC * NS)
NIB = 30
NBUF = 3
SLAG = 1
ROWB = 400
_MESH = plsc.VectorSubcoreMesh(core_axis_name="c", subcore_axis_name="s")
_SC_PARAMS = pltpu.CompilerParams(use_tc_tiling_on_sc=False,
                                  needs_layout_passes=False)

f32 = jnp.float32


def _zero_fill(ref, nrow, ncol):
    nv = ncol // 16

    def body(i, _):
        r = i // nv
        k = i % nv
        ref[r, pl.ds(k * 16, 16)] = jnp.zeros((16,), f32)
        return 0

    lax.fori_loop(0, nrow * nv, body, 0)


def _zero_acc(stage, acc_sh, base):
    off = 0
    left = TROWS
    while left > 0:
        n = min(CHUNK, left)
        pltpu.sync_copy(stage.at[pl.ds(0, n)], acc_sh.at[pl.ds(base + off, n)])
        off += n
        left -= n


@functools.partial(
    pl.kernel,
    out_type=jax.ShapeDtypeStruct((NC * ACCR, 16), f32),
    mesh=_MESH,
    scratch_types=[
        pltpu.VMEM((CH_DEG, CHUNK), jnp.int32),
        pltpu.VMEM((CHUNK, 16), f32),
        pltpu.VMEM_SHARED((ACCR, 16), f32),
        pltpu.SemaphoreType.DMA,
    ],
    compiler_params=_SC_PARAMS,
)
def _deg(dst2d_hbm, out_hbm, dstb, ones_v, acc_sh, sem):
    c = lax.axis_index("c")
    s = lax.axis_index("s")
    w = c * NS + s
    pltpu.sync_copy(dst2d_hbm.at[pl.ds(w * CH_DEG, CH_DEG)], dstb)
    _zero_fill(ones_v, CHUNK, 16)
    base = s * TROWS
    _zero_acc(ones_v, acc_sh, base)

    def fill(i, _):
        ones_v[i, pl.ds(0, 16)] = jnp.full((16,), 1.0, f32)
        return 0

    lax.fori_loop(0, CHUNK, fill, 0)
    plsc.subcore_barrier()

    def fire(j, _):
        pltpu.async_copy(ones_v, acc_sh.at[dstb.at[j]], sem, add=True)
        return 0

    lax.fori_loop(0, CH_DEG, fire, 0)

    def drain(j, _):
        pltpu.make_async_copy(ones_v, acc_sh.at[dstb.at[0]], sem).wait()
        return 0

    lax.fori_loop(0, CH_DEG, drain, 0)
    plsc.subcore_barrier()
    pltpu.sync_copy(acc_sh.at[pl.ds(base, TROWS)],
                    out_hbm.at[pl.ds(c * ACCR + base, TROWS)])


def _make_agg(dc):
    nvw = CHUNK // 16

    @functools.partial(
        pl.kernel,
        out_type=jax.ShapeDtypeStruct((NC * ACCR, dc), f32),
        mesh=_MESH,
        scratch_types=[
            pltpu.VMEM((NIB, CHUNK), jnp.int32),
            pltpu.VMEM((NIB, CHUNK), jnp.int32),
            pltpu.VMEM((CHUNK, dc), f32),
            pltpu.VMEM((CHUNK, dc), f32),
            pltpu.VMEM((CHUNK, dc), f32),
            pltpu.VMEM_SHARED((ACCR, dc), f32),
            pltpu.SemaphoreType.DMA,
            pltpu.SemaphoreType.DMA,
            pltpu.SemaphoreType.DMA,
            pltpu.SemaphoreType.DMA,
        ],
        compiler_params=_SC_PARAMS,
    )
    def agg(xs_hbm, src2d_hbm, dst2d_hbm, out_hbm,
            srcb, dstb, r0, r1, r2, acc_sh, gs0, gs1, gs2, ssem):
        c = lax.axis_index("c")
        s = lax.axis_index("s")
        off = (c * ACCR).astype(jnp.int32)
        _zero_fill(r0, CHUNK, dc)
        base = s * TROWS
        _zero_acc(r0, acc_sh, base)
        plsc.subcore_barrier()

        bufs = (r0, r1, r2)
        gsems = (gs0, gs1, gs2)

        def gather(j, b):
            pltpu.async_copy(xs_hbm.at[srcb.at[j]], bufs[b], gsems[b])

        def gwait(b):
            pltpu.make_async_copy(xs_hbm.at[pl.ds(0, CHUNK)], bufs[b],
                                  gsems[b]).wait()

        def swait():
            pltpu.make_async_copy(r0, acc_sh.at[dstb.at[0]], ssem).wait()

        def block(g, _):
            hb = s * CH_AGG + g * NIB
            pltpu.sync_copy(src2d_hbm.at[pl.ds(hb, NIB)], srcb)
            pltpu.sync_copy(dst2d_hbm.at[pl.ds(hb, NIB)], dstb)

            def addoff(i, __):
                r = i // nvw
                k = i % nvw
                srcb[r, pl.ds(k * 16, 16)] = srcb[r, pl.ds(k * 16, 16)] + off
                return 0

            lax.fori_loop(0, NIB * nvw, addoff, 0)
            gather(0, 0)
            gather(1, 1)
            for j in range(NIB):
                b = j % NBUF
                gwait(b)
                pltpu.async_copy(bufs[b], acc_sh.at[dstb.at[j]], ssem,
                                 add=True)
                if j >= SLAG:
                    swait()
                if j + 2 < NIB:
                    gather(j + 2, (j + 2) % NBUF)
            for _ in range(min(SLAG, NIB)):
                swait()
            return 0

        lax.fori_loop(0, CH_AGG // NIB, block, 0)
        plsc.subcore_barrier()
        pltpu.sync_copy(acc_sh.at[pl.ds(base, TROWS)],
                        out_hbm.at[pl.ds(c * ACCR + base, TROWS)])

    return agg


_agg64 = _make_agg(64)
_agg128 = _make_agg(128)


def _prep_body(p0, p1, x, xs_s, dinv):
    deg = p0[0, :, :1] + p1[0, :, :1] + 1.0
    div = lax.rsqrt(deg)
    xs = x[:, :] * div
    xs_s[0, :, :] = xs[:, : DIN // 2]
    xs_s[1, :, :] = xs[:, DIN // 2:]
    dinv[:, :] = div


_prep = pl.pallas_call(
    _prep_body,
    grid=(NN // ROWB,),
    in_specs=[
        pl.BlockSpec((1, ROWB, 16), lambda i: (0, i, 0)),
        pl.BlockSpec((1, ROWB, 16), lambda i: (1, i, 0)),
        pl.BlockSpec((ROWB, DIN), lambda i: (i, 0)),
    ],
    out_specs=[
        pl.BlockSpec((NC, ROWB, DIN // 2), lambda i: (0, i, 0)),
        pl.BlockSpec((ROWB, 1), lambda i: (i, 0)),
    ],
    out_shape=[
        jax.ShapeDtypeStruct((NC, ACCR, DIN // 2), f32),
        jax.ShapeDtypeStruct((NN, 1), f32),
    ],
)


def _l1_body(a0, a1, x, dinv, W1, b1, g, bn, g2_s):
    agg = jnp.concatenate([a0[0], a1[0]], axis=1)
    div = dinv[:, :]
    t = (agg + x[:, :] * div) * div
    h = jnp.dot(t.astype(jnp.bfloat16), W1[:, :].astype(jnp.bfloat16),
                preferred_element_type=f32) + b1[:, :]
    h = jnp.maximum(h, 0.0)
    mu = jnp.mean(h, axis=1, keepdims=True)
    var = jnp.mean((h - mu) ** 2, axis=1, keepdims=True)
    h = (h - mu) * lax.rsqrt(var + 1e-5) * g[:, :] + bn[:, :]
    h = h * div
    g2_s[0, :, :] = h[:, : DH // 2]
    g2_s[1, :, :] = h[:, DH // 2:]


_l1 = pl.pallas_call(
    _l1_body,
    grid=(NN // ROWB,),
    in_specs=[
        pl.BlockSpec((1, ROWB, DIN // 2), lambda i: (0, i, 0)),
        pl.BlockSpec((1, ROWB, DIN // 2), lambda i: (1, i, 0)),
        pl.BlockSpec((ROWB, DIN), lambda i: (i, 0)),
        pl.BlockSpec((ROWB, 1), lambda i: (i, 0)),
        pl.BlockSpec((DIN, DH), lambda i: (0, 0)),
        pl.BlockSpec((1, DH), lambda i: (0, 0)),
        pl.BlockSpec((1, DH), lambda i: (0, 0)),
        pl.BlockSpec((1, DH), lambda i: (0, 0)),
    ],
    out_specs=[
        pl.BlockSpec((NC, ROWB, DH // 2), lambda i: (0, i, 0)),
    ],
    out_shape=[
        jax.ShapeDtypeStruct((NC, ACCR, DH // 2), f32),
    ],
)


def _l2_body(a0, a1, g0, g1, dinv, W2, b2, l1W, l1b, l2W, l2b, emb, logp):
    agg = jnp.concatenate([a0[0], a1[0]], axis=1)
    g2 = jnp.concatenate([g0[0], g1[0]], axis=1)
    div = dinv[:, :]
    t = (agg + g2) * div
    bf = jnp.bfloat16
    e = jnp.dot(t.astype(bf), W2[:, :].astype(bf),
                preferred_element_type=f32) + b2[:, :]
    emb[:, :] = e
    h = jnp.maximum(e, 0.0)
    h = jnp.dot(h.astype(bf), l1W[:, :].astype(bf),
                preferred_element_type=f32) + l1b[:, :]
    h = jnp.dot(h.astype(bf), l2W[:, :].astype(bf),
                preferred_element_type=f32) + l2b[:, :]
    m = jnp.max(h, axis=1, keepdims=True)
    lse = jnp.log(jnp.sum(jnp.exp(h - m), axis=1, keepdims=True)) + m
    logp[:, :] = h - lse


_l2 = pl.pallas_call(
    _l2_body,
    grid=(NN // ROWB,),
    in_specs=[
        pl.BlockSpec((1, ROWB, DH // 2), lambda i: (0, i, 0)),
        pl.BlockSpec((1, ROWB, DH // 2), lambda i: (1, i, 0)),
        pl.BlockSpec((1, ROWB, DH // 2), lambda i: (0, i, 0)),
        pl.BlockSpec((1, ROWB, DH // 2), lambda i: (1, i, 0)),
        pl.BlockSpec((ROWB, 1), lambda i: (i, 0)),
        pl.BlockSpec((DH, DH), lambda i: (0, 0)),
        pl.BlockSpec((1, DH), lambda i: (0, 0)),
        pl.BlockSpec((DH, DH), lambda i: (0, 0)),
        pl.BlockSpec((1, DH), lambda i: (0, 0)),
        pl.BlockSpec((DH, DOUT), lambda i: (0, 0)),
        pl.BlockSpec((1, DOUT), lambda i: (0, 0)),
    ],
    out_specs=[
        pl.BlockSpec((ROWB, DH), lambda i: (i, 0)),
        pl.BlockSpec((ROWB, DOUT), lambda i: (i, 0)),
    ],
    out_shape=[
        jax.ShapeDtypeStruct((NN, DH), f32),
        jax.ShapeDtypeStruct((NN, DOUT), f32),
    ],
)


def kernel(x, edge_index, W1, b1, W2, b2, ln_g, ln_b, l1_W, l1_b, l2_W, l2_b):
    src = edge_index[0]
    dst = edge_index[1]
    pad = EPAD - EDGES
    srcp = jnp.concatenate(
        [src, jnp.zeros((pad,), src.dtype)]).reshape(NROWCH, CHUNK)
    dstp = jnp.concatenate(
        [dst, jnp.full((pad,), NN, dst.dtype)]).reshape(NROWCH, CHUNK)

    degs = _deg(dstp).reshape(NC, ACCR, 16)
    xs_s, dinv = _prep(degs, degs, x)

    a1 = _agg64(xs_s.reshape(NC * ACCR, DIN // 2), srcp, dstp)
    a1 = a1.reshape(NC, ACCR, DIN // 2)
    g2_s, = _l1(a1, a1, x, dinv, W1, b1.reshape(1, -1),
                ln_g.reshape(1, -1), ln_b.reshape(1, -1))

    a2 = _agg128(g2_s.reshape(NC * ACCR, DH // 2), srcp, dstp)
    a2 = a2.reshape(NC, ACCR, DH // 2)
    emb, logp = _l2(a2, a2, g2_s, g2_s, dinv, W2, b2.reshape(1, -1),
                    l1_W, l1_b.reshape(1, -1), l2_W, l2_b.reshape(1, -1))
    return (emb, logp)


# speedup vs baseline: 1.0008x; 1.0008x over previous
import functools

import jax
import jax.numpy as jnp
from jax import lax
from jax.experimental import pallas as pl
from jax.experimental.pallas import tpu as pltpu
from jax.experimental.pallas import tpu_sc as plsc

NN = 10000
DIN = 128
DH = 256
DOUT = 64
EDGES = 320000
NC, NS = 2, 16
CHUNK = 112
ACCR = 10112
TROWS = ACCR // NS
EPAD = 322560
NROWCH = EPAD // CHUNK
CH_AGG = NROWCH // NS
CH_DEG = NROWCH // (NC * NS)
NIB = 30
NBUF = 3
SLAG = 1
ROWB = 400
_MESH = plsc.VectorSubcoreMesh(core_axis_name="c", subcore_axis_name="s")
_SC_PARAMS = pltpu.CompilerParams(use_tc_tiling_on_sc=False,
                                  needs_layout_passes=False)

f32 = jnp.float32


def _zero_fill(ref, nrow, ncol):
    nv = ncol // 16

    def body(i, _):
        r = i // nv
        k = i % nv
        ref[r, pl.ds(k * 16, 16)] = jnp.zeros((16,), f32)
        return 0

    lax.fori_loop(0, nrow * nv, body, 0)


def _zero_acc(stage, acc_sh, base):
    off = 0
    left = TROWS
    while left > 0:
        n = min(CHUNK, left)
        pltpu.sync_copy(stage.at[pl.ds(0, n)], acc_sh.at[pl.ds(base + off, n)])
        off += n
        left -= n


@functools.partial(
    pl.kernel,
    out_type=jax.ShapeDtypeStruct((NC * ACCR, 16), f32),
    mesh=_MESH,
    scratch_types=[
        pltpu.VMEM((CH_DEG, CHUNK), jnp.int32),
        pltpu.VMEM((CHUNK, 16), f32),
        pltpu.VMEM_SHARED((ACCR, 16), f32),
        pltpu.SemaphoreType.DMA,
    ],
    compiler_params=_SC_PARAMS,
)
def _deg(dst2d_hbm, out_hbm, dstb, ones_v, acc_sh, sem):
    c = lax.axis_index("c")
    s = lax.axis_index("s")
    w = c * NS + s
    pltpu.sync_copy(dst2d_hbm.at[pl.ds(w * CH_DEG, CH_DEG)], dstb)
    _zero_fill(ones_v, CHUNK, 16)
    base = s * TROWS
    _zero_acc(ones_v, acc_sh, base)

    def fill(i, _):
        ones_v[i, pl.ds(0, 16)] = jnp.full((16,), 1.0, f32)
        return 0

    lax.fori_loop(0, CHUNK, fill, 0)
    plsc.subcore_barrier()

    def fire(j, _):
        pltpu.async_copy(ones_v, acc_sh.at[dstb.at[j]], sem, add=True)
        return 0

    lax.fori_loop(0, CH_DEG, fire, 0)

    def drain(j, _):
        pltpu.make_async_copy(ones_v, acc_sh.at[dstb.at[0]], sem).wait()
        return 0

    lax.fori_loop(0, CH_DEG, drain, 0)
    plsc.subcore_barrier()
    pltpu.sync_copy(acc_sh.at[pl.ds(base, TROWS)],
                    out_hbm.at[pl.ds(c * ACCR + base, TROWS)])


def _make_agg(dc):
    nvw = CHUNK // 16

    @functools.partial(
        pl.kernel,
        out_type=jax.ShapeDtypeStruct((NC * ACCR, dc), f32),
        mesh=_MESH,
        scratch_types=[
            pltpu.VMEM((NIB, CHUNK), jnp.int32),
            pltpu.VMEM((NIB, CHUNK), jnp.int32),
            pltpu.VMEM((CHUNK, dc), f32),
            pltpu.VMEM((CHUNK, dc), f32),
            pltpu.VMEM((CHUNK, dc), f32),
            pltpu.VMEM_SHARED((ACCR, dc), f32),
            pltpu.SemaphoreType.DMA,
            pltpu.SemaphoreType.DMA,
            pltpu.SemaphoreType.DMA,
            pltpu.SemaphoreType.DMA,
        ],
        compiler_params=_SC_PARAMS,
    )
    def agg(xs_hbm, src2d_hbm, dst2d_hbm, out_hbm,
            srcb, dstb, r0, r1, r2, acc_sh, gs0, gs1, gs2, ssem):
        c = lax.axis_index("c")
        s = lax.axis_index("s")
        off = (c * ACCR).astype(jnp.int32)
        _zero_fill(r0, CHUNK, dc)
        base = s * TROWS
        _zero_acc(r0, acc_sh, base)
        plsc.subcore_barrier()

        bufs = (r0, r1, r2)
        gsems = (gs0, gs1, gs2)

        def gather(j, b):
            pltpu.async_copy(xs_hbm.at[srcb.at[j]], bufs[b], gsems[b])

        def gwait(b):
            pltpu.make_async_copy(xs_hbm.at[pl.ds(0, CHUNK)], bufs[b],
                                  gsems[b]).wait()

        def swait():
            pltpu.make_async_copy(r0, acc_sh.at[dstb.at[0]], ssem).wait()

        def block(g, _):
            hb = s * CH_AGG + g * NIB
            pltpu.sync_copy(src2d_hbm.at[pl.ds(hb, NIB)], srcb)
            pltpu.sync_copy(dst2d_hbm.at[pl.ds(hb, NIB)], dstb)

            def addoff(i, __):
                r = i // nvw
                k = i % nvw
                srcb[r, pl.ds(k * 16, 16)] = srcb[r, pl.ds(k * 16, 16)] + off
                return 0

            lax.fori_loop(0, NIB * nvw, addoff, 0)
            gather(0, 0)
            gather(1, 1)
            for j in range(NIB):
                b = j % NBUF
                gwait(b)
                pltpu.async_copy(bufs[b], acc_sh.at[dstb.at[j]], ssem,
                                 add=True)
                if j >= SLAG:
                    swait()
                if j + 2 < NIB:
                    gather(j + 2, (j + 2) % NBUF)
            for _ in range(min(SLAG, NIB)):
                swait()
            return 0

        lax.fori_loop(0, CH_AGG // NIB, block, 0)
        plsc.subcore_barrier()
        pltpu.sync_copy(acc_sh.at[pl.ds(base, TROWS)],
                        out_hbm.at[pl.ds(c * ACCR + base, TROWS)])

    return agg


_agg64 = _make_agg(64)
_agg128 = _make_agg(128)


def _prep_body(p0, p1, x, xs_s, dinv):
    deg = p0[0, :, :1] + p1[0, :, :1] + 1.0
    div = lax.rsqrt(deg)
    xs = x[:, :] * div
    xs_s[0, :, :] = xs[:, : DIN // 2]
    xs_s[1, :, :] = xs[:, DIN // 2:]
    dinv[:, :] = div


_prep = pl.pallas_call(
    _prep_body,
    grid=(NN // ROWB,),
    in_specs=[
        pl.BlockSpec((1, ROWB, 16), lambda i: (0, i, 0)),
        pl.BlockSpec((1, ROWB, 16), lambda i: (1, i, 0)),
        pl.BlockSpec((ROWB, DIN), lambda i: (i, 0)),
    ],
    out_specs=[
        pl.BlockSpec((NC, ROWB, DIN // 2), lambda i: (0, i, 0)),
        pl.BlockSpec((ROWB, 1), lambda i: (i, 0)),
    ],
    out_shape=[
        jax.ShapeDtypeStruct((NC, ACCR, DIN // 2), f32),
        jax.ShapeDtypeStruct((NN, 1), f32),
    ],
)


def _l1_body(a0, a1, x, dinv, W1, b1, g, bn, g2_s):
    agg = jnp.concatenate([a0[0], a1[0]], axis=1)
    div = dinv[:, :]
    t = (agg + x[:, :] * div) * div
    h = jnp.dot(t, W1[:, :], preferred_element_type=f32) + b1[:, :]
    h = jnp.maximum(h, 0.0)
    mu = jnp.mean(h, axis=1, keepdims=True)
    var = jnp.mean((h - mu) ** 2, axis=1, keepdims=True)
    h = (h - mu) * lax.rsqrt(var + 1e-5) * g[:, :] + bn[:, :]
    h = h * div
    g2_s[0, :, :] = h[:, : DH // 2]
    g2_s[1, :, :] = h[:, DH // 2:]


_l1 = pl.pallas_call(
    _l1_body,
    grid=(NN // ROWB,),
    in_specs=[
        pl.BlockSpec((1, ROWB, DIN // 2), lambda i: (0, i, 0)),
        pl.BlockSpec((1, ROWB, DIN // 2), lambda i: (1, i, 0)),
        pl.BlockSpec((ROWB, DIN), lambda i: (i, 0)),
        pl.BlockSpec((ROWB, 1), lambda i: (i, 0)),
        pl.BlockSpec((DIN, DH), lambda i: (0, 0)),
        pl.BlockSpec((1, DH), lambda i: (0, 0)),
        pl.BlockSpec((1, DH), lambda i: (0, 0)),
        pl.BlockSpec((1, DH), lambda i: (0, 0)),
    ],
    out_specs=[
        pl.BlockSpec((NC, ROWB, DH // 2), lambda i: (0, i, 0)),
    ],
    out_shape=[
        jax.ShapeDtypeStruct((NC, ACCR, DH // 2), f32),
    ],
)


def _l2_body(a0, a1, g0, g1, dinv, W2, b2, l1W, l1b, l2W, l2b, emb, logp):
    agg = jnp.concatenate([a0[0], a1[0]], axis=1)
    g2 = jnp.concatenate([g0[0], g1[0]], axis=1)
    div = dinv[:, :]
    t = (agg + g2) * div
    e = jnp.dot(t, W2[:, :], preferred_element_type=f32) + b2[:, :]
    emb[:, :] = e
    h = jnp.maximum(e, 0.0)
    h = jnp.dot(h, l1W[:, :], preferred_element_type=f32) + l1b[:, :]
    h = jnp.dot(h, l2W[:, :], preferred_element_type=f32) + l2b[:, :]
    m = jnp.max(h, axis=1, keepdims=True)
    lse = jnp.log(jnp.sum(jnp.exp(h - m), axis=1, keepdims=True)) + m
    logp[:, :] = h - lse


_l2 = pl.pallas_call(
    _l2_body,
    grid=(NN // ROWB,),
    in_specs=[
        pl.BlockSpec((1, ROWB, DH // 2), lambda i: (0, i, 0)),
        pl.BlockSpec((1, ROWB, DH // 2), lambda i: (1, i, 0)),
        pl.BlockSpec((1, ROWB, DH // 2), lambda i: (0, i, 0)),
        pl.BlockSpec((1, ROWB, DH // 2), lambda i: (1, i, 0)),
        pl.BlockSpec((ROWB, 1), lambda i: (i, 0)),
        pl.BlockSpec((DH, DH), lambda i: (0, 0)),
        pl.BlockSpec((1, DH), lambda i: (0, 0)),
        pl.BlockSpec((DH, DH), lambda i: (0, 0)),
        pl.BlockSpec((1, DH), lambda i: (0, 0)),
        pl.BlockSpec((DH, DOUT), lambda i: (0, 0)),
        pl.BlockSpec((1, DOUT), lambda i: (0, 0)),
    ],
    out_specs=[
        pl.BlockSpec((ROWB, DH), lambda i: (i, 0)),
        pl.BlockSpec((ROWB, DOUT), lambda i: (i, 0)),
    ],
    out_shape=[
        jax.ShapeDtypeStruct((NN, DH), f32),
        jax.ShapeDtypeStruct((NN, DOUT), f32),
    ],
)


def kernel(x, edge_index, W1, b1, W2, b2, ln_g, ln_b, l1_W, l1_b, l2_W, l2_b):
    src = edge_index[0]
    dst = edge_index[1]
    pad = EPAD - EDGES
    srcp = jnp.concatenate(
        [src, jnp.zeros((pad,), src.dtype)]).reshape(NROWCH, CHUNK)
    dstp = jnp.concatenate(
        [dst, jnp.full((pad,), NN, dst.dtype)]).reshape(NROWCH, CHUNK)

    degs = _deg(dstp).reshape(NC, ACCR, 16)
    xs_s, dinv = _prep(degs, degs, x)

    a1 = _agg64(xs_s.reshape(NC * ACCR, DIN // 2), srcp, dstp)
    a1 = a1.reshape(NC, ACCR, DIN // 2)
    g2_s, = _l1(a1, a1, x, dinv, W1, b1.reshape(1, -1),
                ln_g.reshape(1, -1), ln_b.reshape(1, -1))

    a2 = _agg128(g2_s.reshape(NC * ACCR, DH // 2), srcp, dstp)
    a2 = a2.reshape(NC, ACCR, DH // 2)
    emb, logp = _l2(a2, a2, g2_s, g2_s, dinv, W2, b2.reshape(1, -1),
                    l1_W, l1_b.reshape(1, -1), l2_W, l2_b.reshape(1, -1))
    return (emb, logp)
